# Initial kernel scaffold; baseline (speedup 1.0000x reference)
import jax
import jax.numpy as jnp
from jax.experimental import pallas as pl


def kernel(x, W):
    raise NotImplementedError("write your pallas kernel here")


# speedup vs baseline: 1.3616x; 1.3616x over previous
import functools

import jax
import jax.numpy as jnp
from jax import lax
from jax.experimental import pallas as pl
from jax.experimental.pallas import tpu as pltpu
from jax.experimental.pallas import tpu_sc as plsc

_TOKENS = 16384
_DIM = 2048
_E = 64
_NC = 2
_NS = 16
_NW = _NC * _NS
_CHUNK = _TOKENS // _NW
_BT = _CHUNK
_LANES = 16


def _tc_body(x_ref, w_ref, out_ref):
    logits = lax.dot_general(
        w_ref[...], x_ref[...],
        dimension_numbers=(((1,), (1,)), ((), ())),
        preferred_element_type=jnp.float32)
    m = jnp.max(logits, axis=0, keepdims=True)
    ex = jnp.exp(logits - m)
    out_ref[0] = ex / jnp.sum(ex, axis=0, keepdims=True)


def _tc_probs_t(x, W):
    return pl.pallas_call(
        _tc_body,
        grid=(_TOKENS // _BT,),
        in_specs=[
            pl.BlockSpec((_BT, _DIM), lambda i: (i, 0)),
            pl.BlockSpec((_E, _DIM), lambda i: (0, 0)),
        ],
        out_specs=pl.BlockSpec((1, _E, _CHUNK), lambda i: (i, 0, 0)),
        out_shape=jax.ShapeDtypeStruct((_NW, _E, _CHUNK), jnp.float32),
    )(x, W)


def _sc_top2(probs_t3):
    mesh = plsc.VectorSubcoreMesh(core_axis_name="c", subcore_axis_name="s")

    @functools.partial(
        pl.kernel, mesh=mesh,
        out_type=[
            jax.ShapeDtypeStruct((_TOKENS,), jnp.float32),
            jax.ShapeDtypeStruct((_TOKENS,), jnp.float32),
            jax.ShapeDtypeStruct((_TOKENS,), jnp.int32),
            jax.ShapeDtypeStruct((_TOKENS,), jnp.int32),
        ],
        scratch_types=[
            pltpu.VMEM((_E, _CHUNK), jnp.float32),
            pltpu.VMEM((_CHUNK,), jnp.float32),
            pltpu.VMEM((_CHUNK,), jnp.float32),
            pltpu.VMEM((_CHUNK,), jnp.int32),
            pltpu.VMEM((_CHUNK,), jnp.int32),
        ],
    )
    def k(probs_hbm, g1_hbm, g2_hbm, i1_hbm, i2_hbm,
          tile_v, g1_v, g2_v, i1_v, i2_v):
        wid = lax.axis_index("s") * _NC + lax.axis_index("c")
        pltpu.sync_copy(probs_hbm.at[wid], tile_v)

        def token_block(v, _):
            off = v * _LANES
            neg = jnp.full((_LANES,), -jnp.inf, jnp.float32)
            iz = jnp.zeros((_LANES,), jnp.int32)

            def expert_step(e, carry):
                m1, i1, m2, i2 = carry
                val = tile_v[e, pl.ds(off, _LANES)]
                ev = jnp.full((_LANES,), e, jnp.int32)
                gt1 = val > m1
                gt2 = val > m2
                nm2 = jnp.where(gt1, m1, jnp.where(gt2, val, m2))
                ni2 = jnp.where(gt1, i1, jnp.where(gt2, ev, i2))
                nm1 = jnp.where(gt1, val, m1)
                ni1 = jnp.where(gt1, ev, i1)
                return nm1, ni1, nm2, ni2

            m1, i1, m2, i2 = lax.fori_loop(
                0, _E, expert_step, (neg, iz, neg, iz))
            s = m1 + m2 + jnp.float32(1e-8)
            g1_v[pl.ds(off, _LANES)] = m1 / s
            g2_v[pl.ds(off, _LANES)] = m2 / s
            i1_v[pl.ds(off, _LANES)] = i1
            i2_v[pl.ds(off, _LANES)] = i2
            return 0

        lax.fori_loop(0, _CHUNK // _LANES, token_block, 0)
        base = wid * _CHUNK
        pltpu.sync_copy(g1_v, g1_hbm.at[pl.ds(base, _CHUNK)])
        pltpu.sync_copy(g2_v, g2_hbm.at[pl.ds(base, _CHUNK)])
        pltpu.sync_copy(i1_v, i1_hbm.at[pl.ds(base, _CHUNK)])
        pltpu.sync_copy(i2_v, i2_hbm.at[pl.ds(base, _CHUNK)])

    return k(probs_t3)


def kernel(x, W):
    probs_t3 = _tc_probs_t(x, W)
    g1, g2, i1, i2 = _sc_top2(probs_t3)
    router_probs = probs_t3.transpose(0, 2, 1).reshape(_TOKENS, _E)
    gate_weights = jnp.stack([g1, g2], axis=-1)
    top_k_indices = jnp.stack([i1, i2], axis=-1)
    return gate_weights, top_k_indices, router_probs
